# baseline (speedup 1.0000x reference)
import functools

import jax
import jax.numpy as jnp
from jax.experimental import pallas as pl

_TEMP = 0.1
_BASE_TEMP = 0.07
_POS_K = 1024
_NEG_K = 2048
_ROW_BLOCK = 80


def _f2k(v):
    b = jax.lax.bitcast_convert_type(v, jnp.int32)
    return b ^ ((b >> 31) & jnp.int32(0x7FFFFFFF))


def _k2f(k):
    b = k ^ ((k >> 31) & jnp.int32(0x7FFFFFFF))
    return jax.lax.bitcast_convert_type(b, jnp.float32)


def _rowsum(x):
    ones = jnp.full((x.shape[1], 1), 1.0, jnp.float32)
    return jax.lax.dot_general(
        x, ones, (((1,), (0,)), ((), ())), preferred_element_type=jnp.float32
    )


def _kth_smallest_keys2(keys_a, k_a, keys_b, k_b):
    rows = keys_a.shape[0]
    imin = jnp.iinfo(jnp.int32).min
    imax = jnp.iinfo(jnp.int32).max

    def one(keys, k, lo, hi):
        mid = (lo & hi) + ((lo ^ hi) >> 1)
        cnt = _rowsum((keys <= mid).astype(jnp.float32))
        ge = cnt >= k
        return jnp.where(ge, lo, mid + 1), jnp.where(ge, mid, hi)

    k_a = jnp.float32(k_a)
    k_b = jnp.float32(k_b)

    def body(_, carry):
        lo_a, hi_a, lo_b, hi_b = carry
        lo_a, hi_a = one(keys_a, k_a, lo_a, hi_a)
        lo_b, hi_b = one(keys_b, k_b, lo_b, hi_b)
        return lo_a, hi_a, lo_b, hi_b

    init = (
        jnp.full((rows, 1), imin, jnp.int32),
        jnp.full((rows, 1), imax, jnp.int32),
        jnp.full((rows, 1), imin, jnp.int32),
        jnp.full((rows, 1), imax, jnp.int32),
    )
    _, hi_a, _, hi_b = jax.lax.fori_loop(0, 32, body, init)
    return hi_a, hi_b


def _norm_body(x_ref, o_ref):
    x = x_ref[...]
    n = jnp.sqrt(jnp.sum(x * x, axis=1, keepdims=True))
    o_ref[...] = x / jnp.maximum(n, 1e-12)


def _loss_body(a_ref, y_ref, c_ref, o_ref, *, class_num, cache_size, total_rows):
    ctot = class_num * cache_size
    a = a_ref[...]
    an = jnp.sqrt(jnp.sum(a * a, axis=1, keepdims=True))
    a = a / jnp.maximum(an, 1e-12)
    dot = jax.lax.dot_general(
        a, c_ref[...], (((1,), (1,)), ((), ())),
        preferred_element_type=jnp.float32,
        precision=jax.lax.Precision.DEFAULT,
    ) / _TEMP
    rows = dot.shape[0]
    y = y_ref[...]

    dot3 = dot.reshape(rows, class_num, cache_size)
    sel3 = jax.lax.broadcasted_iota(jnp.int32, (1, class_num, 1), 1) == y[:, :, None]
    posv = jnp.sum(jnp.where(sel3, dot3, 0.0), axis=1)
    pos_keys = _f2k(posv)

    own = jax.lax.broadcasted_iota(jnp.int32, (1, ctot), 1) // cache_size == y
    negv = jnp.where(own, -jnp.inf, dot)
    neg_keys = _f2k(negv)

    kpos, kneg = _kth_smallest_keys2(pos_keys, _POS_K, neg_keys, ctot - _NEG_K)
    t = _k2f(kpos)
    u = _k2f(kneg)
    m = jnp.maximum(t, jnp.max(negv, axis=1, keepdims=True))

    gt = neg_keys > kneg
    cnt_gt = _rowsum(gt.astype(jnp.float32))
    s1 = _rowsum(jnp.where(gt, jnp.exp(negv - m), 0.0))
    sneg = s1 + (_NEG_K - cnt_gt) * jnp.exp(u - m)

    lt = pos_keys < kpos
    cnt_lt = jnp.sum(lt.astype(jnp.int32), axis=1, keepdims=True)
    g = (posv - m) - jnp.log(jnp.exp(posv - m) + sneg)
    g_t = (t - m) - jnp.log(jnp.exp(t - m) + sneg)
    sum_g = (
        jnp.sum(jnp.where(lt, g, 0.0), axis=1, keepdims=True)
        + (_POS_K - cnt_lt).astype(jnp.float32) * g_t
    )
    row_loss = -(_TEMP / _BASE_TEMP) * (sum_g / _POS_K)
    part = jnp.sum(row_loss, axis=0, keepdims=True) / total_rows

    @pl.when(pl.program_id(0) == 0)
    def _init():
        o_ref[...] = jnp.zeros_like(o_ref)

    o_ref[...] += part


def kernel(X_anchor, y_anchor, queue):
    anchor_num, n_view, feat = X_anchor.shape
    class_num, cache_size, _ = queue.shape
    a_total = anchor_num * n_view
    ctot = class_num * cache_size
    anchors = X_anchor.reshape(a_total, feat)
    contrast = queue.reshape(ctot, feat)
    y_rows = jnp.repeat(y_anchor.astype(jnp.int32), n_view).reshape(a_total, 1)

    nblk = 8
    cn = pl.pallas_call(
        _norm_body,
        grid=(nblk,),
        in_specs=[pl.BlockSpec((ctot // nblk, feat), lambda i: (i, 0))],
        out_specs=pl.BlockSpec((ctot // nblk, feat), lambda i: (i, 0)),
        out_shape=jax.ShapeDtypeStruct((ctot, feat), jnp.float32),
    )(contrast)

    rblk = _ROW_BLOCK if a_total % _ROW_BLOCK == 0 else a_total
    body = functools.partial(
        _loss_body,
        class_num=class_num,
        cache_size=cache_size,
        total_rows=float(a_total),
    )
    out = pl.pallas_call(
        body,
        grid=(a_total // rblk,),
        in_specs=[
            pl.BlockSpec((rblk, feat), lambda i: (i, 0)),
            pl.BlockSpec((rblk, 1), lambda i: (i, 0)),
            pl.BlockSpec((ctot, feat), lambda i: (0, 0)),
        ],
        out_specs=pl.BlockSpec((1, 1), lambda i: (0, 0)),
        out_shape=jax.ShapeDtypeStruct((1, 1), jnp.float32),
    )(anchors, y_rows, cn)
    return out[0, 0]


# speedup vs baseline: 1.0048x; 1.0048x over previous
import functools

import jax
import jax.numpy as jnp
from jax.experimental import pallas as pl

_TEMP = 0.1
_BASE_TEMP = 0.07
_POS_K = 1024
_NEG_K = 2048
_ROW_BLOCK = 80


def _k2f(k):
    b = k ^ ((k >> 31) & jnp.int32(0x7FFFFFFF))
    return jax.lax.bitcast_convert_type(b, jnp.float32)


def _kth_smallest_vals2(vals_a, k_a, vals_b, k_b):
    rows = vals_a.shape[0]
    imin = jnp.iinfo(jnp.int32).min
    imax = jnp.iinfo(jnp.int32).max

    def one(vals, k, lo, hi):
        mid = (lo & hi) + ((lo ^ hi) >> 1)
        cnt = jnp.sum((vals <= _k2f(mid)).astype(jnp.int32), axis=1, keepdims=True)
        ge = cnt >= k
        return jnp.where(ge, lo, mid + 1), jnp.where(ge, mid, hi)

    def body(_, carry):
        lo_a, hi_a, lo_b, hi_b = carry
        lo_a, hi_a = one(vals_a, k_a, lo_a, hi_a)
        lo_b, hi_b = one(vals_b, k_b, lo_b, hi_b)
        return lo_a, hi_a, lo_b, hi_b

    init = (
        jnp.full((rows, 1), imin, jnp.int32),
        jnp.full((rows, 1), imax, jnp.int32),
        jnp.full((rows, 1), imin, jnp.int32),
        jnp.full((rows, 1), imax, jnp.int32),
    )
    _, hi_a, _, hi_b = jax.lax.fori_loop(0, 32, body, init)
    return _k2f(hi_a), _k2f(hi_b)


def _norm_body(x_ref, o_ref):
    x = x_ref[...]
    n = jnp.sqrt(jnp.sum(x * x, axis=1, keepdims=True))
    o_ref[...] = x / jnp.maximum(n, 1e-12)


def _loss_body(a_ref, y_ref, c_ref, o_ref, *, class_num, cache_size, total_rows):
    ctot = class_num * cache_size
    a = a_ref[...]
    an = jnp.sqrt(jnp.sum(a * a, axis=1, keepdims=True))
    a = a / jnp.maximum(an, 1e-12)
    dot = jax.lax.dot_general(
        a, c_ref[...], (((1,), (1,)), ((), ())),
        preferred_element_type=jnp.float32,
        precision=jax.lax.Precision.DEFAULT,
    ) / _TEMP
    rows = dot.shape[0]
    y = y_ref[...]

    dot3 = dot.reshape(rows, class_num, cache_size)
    sel3 = jax.lax.broadcasted_iota(jnp.int32, (1, class_num, 1), 1) == y[:, :, None]
    posv = jnp.sum(jnp.where(sel3, dot3, 0.0), axis=1)

    own = jax.lax.broadcasted_iota(jnp.int32, (1, ctot), 1) // cache_size == y
    negv = jnp.where(own, -jnp.inf, dot)

    t, u = _kth_smallest_vals2(posv, _POS_K, negv, ctot - _NEG_K)
    m = jnp.maximum(t, jnp.max(negv, axis=1, keepdims=True))

    gt = negv > u
    cnt_gt = jnp.sum(gt.astype(jnp.int32), axis=1, keepdims=True)
    s1 = jnp.sum(jnp.where(gt, jnp.exp(negv - m), 0.0), axis=1, keepdims=True)
    sneg = s1 + (_NEG_K - cnt_gt).astype(jnp.float32) * jnp.exp(u - m)

    lt = posv < t
    cnt_lt = jnp.sum(lt.astype(jnp.int32), axis=1, keepdims=True)
    g = (posv - m) - jnp.log(jnp.exp(posv - m) + sneg)
    g_t = (t - m) - jnp.log(jnp.exp(t - m) + sneg)
    sum_g = (
        jnp.sum(jnp.where(lt, g, 0.0), axis=1, keepdims=True)
        + (_POS_K - cnt_lt).astype(jnp.float32) * g_t
    )
    row_loss = -(_TEMP / _BASE_TEMP) * (sum_g / _POS_K)
    part = jnp.sum(row_loss, axis=0, keepdims=True) / total_rows

    @pl.when(pl.program_id(0) == 0)
    def _init():
        o_ref[...] = jnp.zeros_like(o_ref)

    o_ref[...] += part


def kernel(X_anchor, y_anchor, queue):
    anchor_num, n_view, feat = X_anchor.shape
    class_num, cache_size, _ = queue.shape
    a_total = anchor_num * n_view
    ctot = class_num * cache_size
    anchors = X_anchor.reshape(a_total, feat)
    contrast = queue.reshape(ctot, feat)
    y_rows = jnp.repeat(y_anchor.astype(jnp.int32), n_view).reshape(a_total, 1)

    nblk = 8
    cn = pl.pallas_call(
        _norm_body,
        grid=(nblk,),
        in_specs=[pl.BlockSpec((ctot // nblk, feat), lambda i: (i, 0))],
        out_specs=pl.BlockSpec((ctot // nblk, feat), lambda i: (i, 0)),
        out_shape=jax.ShapeDtypeStruct((ctot, feat), jnp.float32),
    )(contrast)

    rblk = _ROW_BLOCK if a_total % _ROW_BLOCK == 0 else a_total
    body = functools.partial(
        _loss_body,
        class_num=class_num,
        cache_size=cache_size,
        total_rows=float(a_total),
    )
    out = pl.pallas_call(
        body,
        grid=(a_total // rblk,),
        in_specs=[
            pl.BlockSpec((rblk, feat), lambda i: (i, 0)),
            pl.BlockSpec((rblk, 1), lambda i: (i, 0)),
            pl.BlockSpec((ctot, feat), lambda i: (0, 0)),
        ],
        out_specs=pl.BlockSpec((1, 1), lambda i: (0, 0)),
        out_shape=jax.ShapeDtypeStruct((1, 1), jnp.float32),
    )(anchors, y_rows, cn)
    return out[0, 0]


# speedup vs baseline: 1.0532x; 1.0482x over previous
import functools

import jax
import jax.numpy as jnp
from jax.experimental import pallas as pl

_TEMP = 0.1
_BASE_TEMP = 0.07
_POS_K = 1024
_NEG_K = 2048
_ROW_BLOCK = 160


def _k2f(k):
    b = k ^ ((k >> 31) & jnp.int32(0x7FFFFFFF))
    return jax.lax.bitcast_convert_type(b, jnp.float32)


def _kth_smallest_vals2(vals_a, k_a, vals_b, k_b):
    rows = vals_a.shape[0]
    imin = jnp.iinfo(jnp.int32).min
    imax = jnp.iinfo(jnp.int32).max

    def one(vals, k, lo, hi):
        mid = (lo & hi) + ((lo ^ hi) >> 1)
        cnt = jnp.sum((vals <= _k2f(mid)).astype(jnp.int32), axis=1, keepdims=True)
        ge = cnt >= k
        return jnp.where(ge, lo, mid + 1), jnp.where(ge, mid, hi)

    def body(_, carry):
        lo_a, hi_a, lo_b, hi_b = carry
        lo_a, hi_a = one(vals_a, k_a, lo_a, hi_a)
        lo_b, hi_b = one(vals_b, k_b, lo_b, hi_b)
        return lo_a, hi_a, lo_b, hi_b

    init = (
        jnp.full((rows, 1), imin, jnp.int32),
        jnp.full((rows, 1), imax, jnp.int32),
        jnp.full((rows, 1), imin, jnp.int32),
        jnp.full((rows, 1), imax, jnp.int32),
    )
    _, hi_a, _, hi_b = jax.lax.fori_loop(0, 32, body, init)
    return _k2f(hi_a), _k2f(hi_b)


def _norm_body(x_ref, o_ref):
    x = x_ref[...]
    n = jnp.sqrt(jnp.sum(x * x, axis=1, keepdims=True))
    o_ref[...] = x / jnp.maximum(n, 1e-12)


def _loss_body(a_ref, y_ref, c_ref, o_ref, *, class_num, cache_size, total_rows):
    ctot = class_num * cache_size
    a = a_ref[...]
    an = jnp.sqrt(jnp.sum(a * a, axis=1, keepdims=True))
    a = a / jnp.maximum(an, 1e-12)
    dot = jax.lax.dot_general(
        a, c_ref[...], (((1,), (1,)), ((), ())),
        preferred_element_type=jnp.float32,
        precision=jax.lax.Precision.DEFAULT,
    ) / _TEMP
    rows = dot.shape[0]
    y = y_ref[...]

    dot3 = dot.reshape(rows, class_num, cache_size)
    sel3 = jax.lax.broadcasted_iota(jnp.int32, (1, class_num, 1), 1) == y[:, :, None]
    posv = jnp.sum(jnp.where(sel3, dot3, 0.0), axis=1)

    own = jax.lax.broadcasted_iota(jnp.int32, (1, ctot), 1) // cache_size == y
    negv = jnp.where(own, -jnp.inf, dot)

    t, u = _kth_smallest_vals2(posv, _POS_K, negv, ctot - _NEG_K)
    m = jnp.maximum(t, jnp.max(negv, axis=1, keepdims=True))

    gt = negv > u
    cnt_gt = jnp.sum(gt.astype(jnp.int32), axis=1, keepdims=True)
    s1 = jnp.sum(jnp.where(gt, jnp.exp(negv - m), 0.0), axis=1, keepdims=True)
    sneg = s1 + (_NEG_K - cnt_gt).astype(jnp.float32) * jnp.exp(u - m)

    lt = posv < t
    cnt_lt = jnp.sum(lt.astype(jnp.int32), axis=1, keepdims=True)
    g = (posv - m) - jnp.log(jnp.exp(posv - m) + sneg)
    g_t = (t - m) - jnp.log(jnp.exp(t - m) + sneg)
    sum_g = (
        jnp.sum(jnp.where(lt, g, 0.0), axis=1, keepdims=True)
        + (_POS_K - cnt_lt).astype(jnp.float32) * g_t
    )
    row_loss = -(_TEMP / _BASE_TEMP) * (sum_g / _POS_K)
    part = jnp.sum(row_loss, axis=0, keepdims=True) / total_rows

    @pl.when(pl.program_id(0) == 0)
    def _init():
        o_ref[...] = jnp.zeros_like(o_ref)

    o_ref[...] += part


def kernel(X_anchor, y_anchor, queue):
    anchor_num, n_view, feat = X_anchor.shape
    class_num, cache_size, _ = queue.shape
    a_total = anchor_num * n_view
    ctot = class_num * cache_size
    anchors = X_anchor.reshape(a_total, feat)
    contrast = queue.reshape(ctot, feat)
    y_rows = jnp.repeat(y_anchor.astype(jnp.int32), n_view).reshape(a_total, 1)

    nblk = 8
    cn = pl.pallas_call(
        _norm_body,
        grid=(nblk,),
        in_specs=[pl.BlockSpec((ctot // nblk, feat), lambda i: (i, 0))],
        out_specs=pl.BlockSpec((ctot // nblk, feat), lambda i: (i, 0)),
        out_shape=jax.ShapeDtypeStruct((ctot, feat), jnp.float32),
    )(contrast)

    rblk = _ROW_BLOCK if a_total % _ROW_BLOCK == 0 else a_total
    body = functools.partial(
        _loss_body,
        class_num=class_num,
        cache_size=cache_size,
        total_rows=float(a_total),
    )
    out = pl.pallas_call(
        body,
        grid=(a_total // rblk,),
        in_specs=[
            pl.BlockSpec((rblk, feat), lambda i: (i, 0)),
            pl.BlockSpec((rblk, 1), lambda i: (i, 0)),
            pl.BlockSpec((ctot, feat), lambda i: (0, 0)),
        ],
        out_specs=pl.BlockSpec((1, 1), lambda i: (0, 0)),
        out_shape=jax.ShapeDtypeStruct((1, 1), jnp.float32),
    )(anchors, y_rows, cn)
    return out[0, 0]


# speedup vs baseline: 1.6741x; 1.5896x over previous
import functools

import jax
import jax.numpy as jnp
from jax.experimental import pallas as pl

_TEMP = 0.1
_BASE_TEMP = 0.07
_POS_K = 1024
_NEG_K = 2048
_ROW_BLOCK = 200


def _k2f(k):
    b = k ^ ((k >> 31) & jnp.int32(0x7FFFFFFF))
    return jax.lax.bitcast_convert_type(b, jnp.float32)


def _kth_smallest_vals2(vals_a, k_a, vals_b, k_b):
    rows = vals_a.shape[0]
    imin = jnp.iinfo(jnp.int32).min
    imax = jnp.iinfo(jnp.int32).max

    def one(vals, k, lo, hi):
        mid = (lo & hi) + ((lo ^ hi) >> 1)
        cnt = jnp.sum((vals <= _k2f(mid)).astype(jnp.int32), axis=1, keepdims=True)
        ge = cnt >= k
        return jnp.where(ge, lo, mid + 1), jnp.where(ge, mid, hi)

    def body(_, carry):
        lo_a, hi_a, lo_b, hi_b = carry
        lo_a, hi_a = one(vals_a, k_a, lo_a, hi_a)
        lo_b, hi_b = one(vals_b, k_b, lo_b, hi_b)
        return lo_a, hi_a, lo_b, hi_b

    init = (
        jnp.full((rows, 1), imin, jnp.int32),
        jnp.full((rows, 1), imax, jnp.int32),
        jnp.full((rows, 1), imin, jnp.int32),
        jnp.full((rows, 1), imax, jnp.int32),
    )
    _, hi_a, _, hi_b = jax.lax.fori_loop(0, 32, body, init)
    return _k2f(hi_a), _k2f(hi_b)


def _norm_body(x_ref, o_ref):
    x = x_ref[...]
    n = jnp.sqrt(jnp.sum(x * x, axis=1, keepdims=True))
    o_ref[...] = x / jnp.maximum(n, 1e-12)


def _loss_body(a_ref, y_ref, c_ref, o_ref, *, class_num, cache_size, total_rows):
    ctot = class_num * cache_size
    a = a_ref[...]
    an = jnp.sqrt(jnp.sum(a * a, axis=1, keepdims=True))
    a = a / jnp.maximum(an, 1e-12)
    dot = jax.lax.dot_general(
        a, c_ref[...], (((1,), (0,)), ((), ())),
        preferred_element_type=jnp.float32,
        precision=jax.lax.Precision.DEFAULT,
    ) / _TEMP
    rows = dot.shape[0]
    y = y_ref[...]

    posv = jnp.zeros((rows, cache_size), jnp.float32)
    for c in range(class_num):
        chunk = dot[:, c * cache_size:(c + 1) * cache_size]
        posv = posv + jnp.where(y == c, chunk, 0.0)

    own = jax.lax.broadcasted_iota(jnp.int32, (1, ctot), 1) // cache_size == y
    negv = jnp.where(own, -jnp.inf, dot)

    t, u = _kth_smallest_vals2(posv, _POS_K, negv, ctot - _NEG_K)
    m = jnp.maximum(t, jnp.max(negv, axis=1, keepdims=True))

    gt = negv > u
    cnt_gt = jnp.sum(gt.astype(jnp.int32), axis=1, keepdims=True)
    s1 = jnp.sum(jnp.where(gt, jnp.exp(negv - m), 0.0), axis=1, keepdims=True)
    sneg = s1 + (_NEG_K - cnt_gt).astype(jnp.float32) * jnp.exp(u - m)

    lt = posv < t
    cnt_lt = jnp.sum(lt.astype(jnp.int32), axis=1, keepdims=True)
    g = (posv - m) - jnp.log(jnp.exp(posv - m) + sneg)
    g_t = (t - m) - jnp.log(jnp.exp(t - m) + sneg)
    sum_g = (
        jnp.sum(jnp.where(lt, g, 0.0), axis=1, keepdims=True)
        + (_POS_K - cnt_lt).astype(jnp.float32) * g_t
    )
    row_loss = -(_TEMP / _BASE_TEMP) * (sum_g / _POS_K)
    part = jnp.sum(row_loss, axis=0, keepdims=True) / total_rows

    @pl.when(pl.program_id(0) == 0)
    def _init():
        o_ref[...] = jnp.zeros_like(o_ref)

    o_ref[...] += part


def kernel(X_anchor, y_anchor, queue):
    anchor_num, n_view, feat = X_anchor.shape
    class_num, cache_size, _ = queue.shape
    a_total = anchor_num * n_view
    ctot = class_num * cache_size
    anchors = X_anchor.reshape(a_total, feat)
    contrast = queue.reshape(ctot, feat)
    y_rows = jnp.repeat(y_anchor.astype(jnp.int32), n_view).reshape(a_total, 1)

    nblk = 8
    cn = pl.pallas_call(
        _norm_body,
        grid=(nblk,),
        in_specs=[pl.BlockSpec((ctot // nblk, feat), lambda i: (i, 0))],
        out_specs=pl.BlockSpec((ctot // nblk, feat), lambda i: (i, 0)),
        out_shape=jax.ShapeDtypeStruct((ctot, feat), jnp.float32),
    )(contrast)

    cnt_t = cn.T

    rblk = _ROW_BLOCK if a_total % _ROW_BLOCK == 0 else a_total
    body = functools.partial(
        _loss_body,
        class_num=class_num,
        cache_size=cache_size,
        total_rows=float(a_total),
    )
    out = pl.pallas_call(
        body,
        grid=(a_total // rblk,),
        in_specs=[
            pl.BlockSpec((rblk, feat), lambda i: (i, 0)),
            pl.BlockSpec((rblk, 1), lambda i: (i, 0)),
            pl.BlockSpec((feat, ctot), lambda i: (0, 0)),
        ],
        out_specs=pl.BlockSpec((1, 1), lambda i: (0, 0)),
        out_shape=jax.ShapeDtypeStruct((1, 1), jnp.float32),
    )(anchors, y_rows, cnt_t)
    return out[0, 0]
